# baseline (speedup 1.0000x reference)
import jax
import jax.numpy as jnp
from jax.experimental import pallas as pl
from jax.experimental.pallas import tpu as pltpu

_ROWS = 250_000
_STREAMS = 10
_GRID = 25
_BLK = _ROWS // (_STREAMS * _GRID)


def _norm_sum_body(*refs):
    o_ref = refs[-1]
    in_refs = refs[:-1]
    step = pl.program_id(0)

    lane = jax.lax.broadcasted_iota(jnp.int32, (128, 4), 0)
    seg = jax.lax.broadcasted_iota(jnp.int32, (128, 4), 1)
    S = (lane // 32 == seg).astype(jnp.float32)

    part = 0.0
    for r in in_refs:
        x = r[...]
        x2 = x * x
        n2 = jax.lax.dot_general(
            x2, S, (((1,), (0,)), ((), ())),
            preferred_element_type=jnp.float32)
        part += jnp.sum(jnp.sqrt(n2))

    @pl.when(step == 0)
    def _init():
        o_ref[0, 0] = 0.0

    o_ref[0, 0] += part


def _spec(s):
    return pl.BlockSpec((_BLK, 128), lambda i, s=s: (s * _GRID + i, 0))


def kernel(user_embedding, item_embedding):
    u = user_embedding.reshape(_ROWS, 128)
    v = item_embedding.reshape(_ROWS, 128)
    args = [u] * _STREAMS + [v] * _STREAMS
    specs = [_spec(s) for s in range(_STREAMS)] * 2
    out = pl.pallas_call(
        _norm_sum_body,
        grid=(_GRID,),
        in_specs=specs,
        out_specs=pl.BlockSpec(memory_space=pltpu.SMEM),
        out_shape=jax.ShapeDtypeStruct((1, 1), jnp.float32),
    )(*args)
    return out[0, 0]


# speedup vs baseline: 1.0439x; 1.0439x over previous
import jax
import jax.numpy as jnp
from jax.experimental import pallas as pl
from jax.experimental.pallas import tpu as pltpu

_N = 1_000_000
_BLK = 8_000
_GRID = _N // _BLK


def _norm_sum_body(u_ref, v_ref, o_ref):
    step = pl.program_id(0)

    ones_col = jnp.ones((32, 1), jnp.float32)
    ones_row = jnp.ones((1, _BLK), jnp.float32)

    def block_total(x):
        x2 = x * x
        n2 = jax.lax.dot_general(
            x2, ones_col, (((1,), (0,)), ((), ())),
            preferred_element_type=jnp.float32)
        s = jnp.sqrt(n2)
        tot = jax.lax.dot_general(
            ones_row, s, (((1,), (0,)), ((), ())),
            preferred_element_type=jnp.float32)
        return tot[0, 0]

    part = block_total(u_ref[...]) + block_total(v_ref[...])

    @pl.when(step == 0)
    def _init():
        o_ref[0, 0] = 0.0

    o_ref[0, 0] += part


def kernel(user_embedding, item_embedding):
    out = pl.pallas_call(
        _norm_sum_body,
        grid=(_GRID,),
        in_specs=[
            pl.BlockSpec((_BLK, 32), lambda i: (i, 0)),
            pl.BlockSpec((_BLK, 32), lambda i: (i, 0)),
        ],
        out_specs=pl.BlockSpec(memory_space=pltpu.SMEM),
        out_shape=jax.ShapeDtypeStruct((1, 1), jnp.float32),
    )(user_embedding, item_embedding)
    return out[0, 0]


# speedup vs baseline: 7.7512x; 7.4255x over previous
import jax
import jax.numpy as jnp
from jax.experimental import pallas as pl
from jax.experimental.pallas import tpu as pltpu

_N = 1_000_000
_CBLK = 8_192
_GRID = -(-_N // _CBLK)


def _norm_sum_body(u_ref, v_ref, o_ref, acc_ref):
    step = pl.program_id(0)

    @pl.when(step == 0)
    def _init():
        acc_ref[...] = jnp.zeros_like(acc_ref)

    ones_row = jnp.ones((1, 8), jnp.float32)
    col = jax.lax.broadcasted_iota(jnp.int32, (1, _CBLK), 1) + step * _CBLK
    valid = col < _N

    def block_norms(x):
        x2 = x * x
        z = x2[0:8, :] + x2[8:16, :] + x2[16:24, :] + x2[24:32, :]
        n2 = jax.lax.dot_general(
            ones_row, z, (((1,), (0,)), ((), ())),
            preferred_element_type=jnp.float32)
        return jnp.where(valid, jnp.sqrt(n2), 0.0)

    acc_ref[0:1, :] += block_norms(u_ref[...]) + block_norms(v_ref[...])

    @pl.when(step == _GRID - 1)
    def _fin():
        o_ref[0, 0] = jnp.sum(acc_ref[0:1, :])


def kernel(user_embedding, item_embedding):
    ut = user_embedding.T
    vt = item_embedding.T
    out = pl.pallas_call(
        _norm_sum_body,
        grid=(_GRID,),
        in_specs=[
            pl.BlockSpec((32, _CBLK), lambda i: (0, i)),
            pl.BlockSpec((32, _CBLK), lambda i: (0, i)),
        ],
        out_specs=pl.BlockSpec(memory_space=pltpu.SMEM),
        out_shape=jax.ShapeDtypeStruct((1, 1), jnp.float32),
        scratch_shapes=[pltpu.VMEM((8, _CBLK), jnp.float32)],
    )(ut, vt)
    return out[0, 0]


# speedup vs baseline: 10.3932x; 1.3409x over previous
import jax
import jax.numpy as jnp
from jax.experimental import pallas as pl
from jax.experimental.pallas import tpu as pltpu

_N = 1_000_000
_CBLK = 16_384
_GRID = -(-_N // _CBLK)


def _norm_sum_body(u_ref, v_ref, o_ref, acc_ref):
    step = pl.program_id(0)

    @pl.when(step == 0)
    def _init():
        acc_ref[...] = jnp.zeros_like(acc_ref)

    ones_row = jnp.ones((1, 8), jnp.float32)
    col = jax.lax.broadcasted_iota(jnp.int32, (1, _CBLK), 1) + step * _CBLK
    valid = col < _N

    def block_norms(x):
        x2 = x * x
        z = x2[0:8, :] + x2[8:16, :] + x2[16:24, :] + x2[24:32, :]
        n2 = jax.lax.dot_general(
            ones_row, z, (((1,), (0,)), ((), ())),
            preferred_element_type=jnp.float32)
        return jnp.where(valid, jnp.sqrt(n2), 0.0)

    acc_ref[0:1, :] += block_norms(u_ref[...]) + block_norms(v_ref[...])

    @pl.when(step == _GRID - 1)
    def _fin():
        o_ref[0, 0] = jnp.sum(acc_ref[0:1, :])


def kernel(user_embedding, item_embedding):
    ut = user_embedding.T
    vt = item_embedding.T
    out = pl.pallas_call(
        _norm_sum_body,
        grid=(_GRID,),
        in_specs=[
            pl.BlockSpec((32, _CBLK), lambda i: (0, i)),
            pl.BlockSpec((32, _CBLK), lambda i: (0, i)),
        ],
        out_specs=pl.BlockSpec(memory_space=pltpu.SMEM),
        out_shape=jax.ShapeDtypeStruct((1, 1), jnp.float32),
        scratch_shapes=[pltpu.VMEM((8, _CBLK), jnp.float32)],
    )(ut, vt)
    return out[0, 0]


# speedup vs baseline: 12.4446x; 1.1974x over previous
import jax
import jax.numpy as jnp
from jax.experimental import pallas as pl
from jax.experimental.pallas import tpu as pltpu

_N = 1_000_000
_CBLK = 32_768
_GRID = -(-_N // _CBLK)


def _norm_sum_body(u_ref, v_ref, o_ref, acc_ref):
    step = pl.program_id(0)

    @pl.when(step == 0)
    def _init():
        acc_ref[...] = jnp.zeros_like(acc_ref)

    ones_row = jnp.ones((1, 8), jnp.float32)
    col = jax.lax.broadcasted_iota(jnp.int32, (1, _CBLK), 1) + step * _CBLK
    valid = col < _N

    def block_norms(x):
        x2 = x * x
        z = x2[0:8, :] + x2[8:16, :] + x2[16:24, :] + x2[24:32, :]
        n2 = jax.lax.dot_general(
            ones_row, z, (((1,), (0,)), ((), ())),
            preferred_element_type=jnp.float32)
        return jnp.where(valid, jnp.sqrt(n2), 0.0)

    acc_ref[0:1, :] += block_norms(u_ref[...]) + block_norms(v_ref[...])

    @pl.when(step == _GRID - 1)
    def _fin():
        o_ref[0, 0] = jnp.sum(acc_ref[0:1, :])


def kernel(user_embedding, item_embedding):
    ut = user_embedding.T
    vt = item_embedding.T
    out = pl.pallas_call(
        _norm_sum_body,
        grid=(_GRID,),
        in_specs=[
            pl.BlockSpec((32, _CBLK), lambda i: (0, i)),
            pl.BlockSpec((32, _CBLK), lambda i: (0, i)),
        ],
        out_specs=pl.BlockSpec(memory_space=pltpu.SMEM),
        out_shape=jax.ShapeDtypeStruct((1, 1), jnp.float32),
        scratch_shapes=[pltpu.VMEM((8, _CBLK), jnp.float32)],
    )(ut, vt)
    return out[0, 0]
